# Initial kernel scaffold; baseline (speedup 1.0000x reference)
import jax
import jax.numpy as jnp
from jax.experimental import pallas as pl


def kernel(node_features, edge_features, edge_indices, edge_indices_reverse, params):
    raise NotImplementedError("write your pallas kernel here")


# speedup vs baseline: 3.5235x; 3.5235x over previous
import functools

import jax
import jax.numpy as jnp
from jax import lax
from jax.experimental import pallas as pl
from jax.experimental.pallas import tpu as pltpu
from jax.experimental.pallas import tpu_sc as plsc

N = 10000
E = 320000
D_NODE = 128
K = 2
D_EDGE = 16
UNITS = 64
EPS = 1e-6

CH = 128
NUM_CHUNKS = E // CH
NUM_TILES = 16
N_PAD = 10240
ROWS_PER_TILE = N_PAD // NUM_TILES


def _rms(x, scale):
  ms = jnp.mean(jnp.square(x), axis=-1, keepdims=True)
  return x * lax.rsqrt(ms + EPS) * scale


def _edge_prep_body(e_ref, w1_ref, b1_ref, w2_ref, b2_ref, rs_ref,
                    ef0_ref, ef1_ref):
  a = e_ref[:, 0, :]
  b = e_ref[:, 1, :]
  ef0_ref[...] = 0.5 * (a + b)

  def branch(x):
    h = jnp.maximum(jnp.dot(x, w1_ref[...],
                            preferred_element_type=jnp.float32) + b1_ref[...],
                    0.0)
    h = jnp.maximum(jnp.dot(h, w2_ref[...],
                            preferred_element_type=jnp.float32) + b2_ref[...],
                    0.0)
    return _rms(h, rs_ref[...])

  ef1_ref[...] = 0.5 * (branch(a) + branch(b))


def _edge_prep(ef3d, w1, b1, w2, b2, rs):
  Be = 2000
  grid = (E // Be,)
  full = lambda i: (0, 0)
  return pl.pallas_call(
      _edge_prep_body,
      grid=grid,
      in_specs=[
          pl.BlockSpec((Be, K, D_EDGE), lambda i: (i, 0, 0)),
          pl.BlockSpec((D_EDGE, UNITS), full),
          pl.BlockSpec((1, UNITS), full),
          pl.BlockSpec((UNITS, UNITS), full),
          pl.BlockSpec((1, UNITS), full),
          pl.BlockSpec((1, UNITS), full),
      ],
      out_specs=[
          pl.BlockSpec((Be, D_EDGE), lambda i: (i, 0)),
          pl.BlockSpec((Be, UNITS), lambda i: (i, 0)),
      ],
      out_shape=[
          jax.ShapeDtypeStruct((E, D_EDGE), jnp.float32),
          jax.ShapeDtypeStruct((E, UNITS), jnp.float32),
      ],
  )(ef3d, w1, b1, w2, b2, rs)


def _proj_body(n_ref, wf_ref, wb_ref, out_ref):
  x = n_ref[...]
  out_ref[0] = jnp.dot(x, wf_ref[...], preferred_element_type=jnp.float32)
  out_ref[1] = jnp.dot(x, wb_ref[...], preferred_element_type=jnp.float32)


def _proj(nodes, wf, wb):
  Bn = 1000
  grid = (N // Bn,)
  full = lambda i: (0, 0)
  return pl.pallas_call(
      _proj_body,
      grid=grid,
      in_specs=[
          pl.BlockSpec((Bn, D_NODE), lambda i: (i, 0)),
          pl.BlockSpec((D_NODE, UNITS), full),
          pl.BlockSpec((D_NODE, UNITS), full),
      ],
      out_specs=pl.BlockSpec((2, Bn, UNITS), lambda i: (0, i, 0)),
      out_shape=jax.ShapeDtypeStruct((2, N, UNITS), jnp.float32),
  )(nodes, wf, wb)


def _sc_body(ed, p_hbm, ef_hbm, src_hbm, dst_hbm, zn_hbm, ze_hbm,
             out_n, out_e, acc_n, acc_e, idx_s, idx_d, rows, ebuf, sem):
  c = lax.axis_index("c")
  s = lax.axis_index("s")

  r0 = s * ROWS_PER_TILE
  pltpu.sync_copy(zn_hbm.at[pl.ds(r0, ROWS_PER_TILE)],
                  acc_n.at[pl.ds(r0, ROWS_PER_TILE)])
  pltpu.sync_copy(ze_hbm.at[pl.ds(r0, ROWS_PER_TILE)],
                  acc_e.at[pl.ds(r0, ROWS_PER_TILE)])
  plsc.subcore_barrier()

  n_iter = (NUM_CHUNKS + NUM_TILES - 1) // NUM_TILES

  def body(k, _):
    j = s + k * NUM_TILES

    @pl.when(j < NUM_CHUNKS)
    def _():
      base = c * E + j * CH
      pltpu.sync_copy(src_hbm.at[pl.ds(base, CH)], idx_s)
      pltpu.sync_copy(dst_hbm.at[pl.ds(base, CH)], idx_d)
      pltpu.async_copy(p_hbm.at[idx_s], rows, sem).wait()
      pltpu.sync_copy(ef_hbm.at[pl.ds(j * CH, CH)], ebuf)
      pltpu.sync_copy(rows, acc_n.at[idx_d], add=True)
      pltpu.sync_copy(ebuf, acc_e.at[idx_d], add=True)

    return ()

  lax.fori_loop(0, n_iter, body, ())
  plsc.subcore_barrier()

  pltpu.sync_copy(acc_n.at[pl.ds(r0, ROWS_PER_TILE)],
                  out_n.at[pl.ds(c * N_PAD + r0, ROWS_PER_TILE)])
  pltpu.sync_copy(acc_e.at[pl.ds(r0, ROWS_PER_TILE)],
                  out_e.at[pl.ds(c * N_PAD + r0, ROWS_PER_TILE)])


def _sc_segsum(ed, p_cat, ef, src_off, dst):
  mesh = plsc.VectorSubcoreMesh(core_axis_name="c", subcore_axis_name="s")
  zn = jnp.zeros((N_PAD, UNITS), jnp.float32)
  ze = jnp.zeros((N_PAD, ed), jnp.float32)
  run = pl.kernel(
      functools.partial(_sc_body, ed),
      out_type=[
          jax.ShapeDtypeStruct((2 * N_PAD, UNITS), jnp.float32),
          jax.ShapeDtypeStruct((2 * N_PAD, ed), jnp.float32),
      ],
      mesh=mesh,
      scratch_types=[
          pltpu.VMEM_SHARED((N_PAD, UNITS), jnp.float32),
          pltpu.VMEM_SHARED((N_PAD, ed), jnp.float32),
          pltpu.VMEM((CH,), jnp.int32),
          pltpu.VMEM((CH,), jnp.int32),
          pltpu.VMEM((CH, UNITS), jnp.float32),
          pltpu.VMEM((CH, ed), jnp.float32),
          pltpu.SemaphoreType.DMA,
      ],
      compiler_params=pltpu.CompilerParams(use_tc_tiling_on_sc=False),
  )
  sn, se = run(p_cat, ef, src_off, dst, zn, ze)
  sn = sn.reshape(2, N_PAD, UNITS)[:, :N, :]
  se = se.reshape(2, N_PAD, ed)[:, :N, :]
  return sn, se


def _node0_body(n_ref, sn_ref, se_ref,
                wft_ref, b1f_ref, w2f_ref, b2f_ref,
                wbt_ref, b1b_ref, w2b_ref, b2b_ref,
                wna_ref, wnb_ref, b1n_ref, w2n_ref, b2n_ref,
                rs_ref, fs_ref, wpf_ref, wpb_ref,
                repr0_ref, nodes1_ref, p1_ref):
  x = n_ref[...]

  def msg(k, wt, b1, w2, b2):
    lin = jnp.dot(se_ref[k], wt[...],
                  preferred_element_type=jnp.float32) + sn_ref[k] + b1[...]
    h = jnp.maximum(lin, 0.0)
    return jnp.maximum(jnp.dot(h, w2[...],
                               preferred_element_type=jnp.float32) + b2[...], 0.0)

  comb = (msg(0, wft_ref, b1f_ref, w2f_ref, b2f_ref) +
          msg(1, wbt_ref, b1b_ref, w2b_ref, b2b_ref))
  h = jnp.maximum(
      jnp.dot(x, wna_ref[...], preferred_element_type=jnp.float32) +
      jnp.dot(comb, wnb_ref[...], preferred_element_type=jnp.float32) +
      b1n_ref[...], 0.0)
  nn = jnp.maximum(jnp.dot(h, w2n_ref[...],
                           preferred_element_type=jnp.float32) + b2n_ref[...], 0.0)
  r0 = _rms(nn, rs_ref[...])
  n1 = _rms(r0, fs_ref[...])
  repr0_ref[...] = r0
  nodes1_ref[...] = n1
  p1_ref[0] = jnp.dot(n1, wpf_ref[...], preferred_element_type=jnp.float32)
  p1_ref[1] = jnp.dot(n1, wpb_ref[...], preferred_element_type=jnp.float32)


def _node_stage0(nodes, sn, se, wft, b1f, w2f, b2f, wbt, b1b, w2b, b2b,
                 wna, wnb, b1n, w2n, b2n, rs, fs, wpf, wpb):
  Bn = 1000
  grid = (N // Bn,)
  full = lambda i: (0, 0)
  U = UNITS
  return pl.pallas_call(
      _node0_body,
      grid=grid,
      in_specs=[
          pl.BlockSpec((Bn, D_NODE), lambda i: (i, 0)),
          pl.BlockSpec((2, Bn, U), lambda i: (0, i, 0)),
          pl.BlockSpec((2, Bn, D_EDGE), lambda i: (0, i, 0)),
          pl.BlockSpec((D_EDGE, U), full),
          pl.BlockSpec((1, U), full),
          pl.BlockSpec((U, U), full),
          pl.BlockSpec((1, U), full),
          pl.BlockSpec((D_EDGE, U), full),
          pl.BlockSpec((1, U), full),
          pl.BlockSpec((U, U), full),
          pl.BlockSpec((1, U), full),
          pl.BlockSpec((D_NODE, U), full),
          pl.BlockSpec((U, U), full),
          pl.BlockSpec((1, U), full),
          pl.BlockSpec((U, U), full),
          pl.BlockSpec((1, U), full),
          pl.BlockSpec((1, U), full),
          pl.BlockSpec((1, U), full),
          pl.BlockSpec((U, U), full),
          pl.BlockSpec((U, U), full),
      ],
      out_specs=[
          pl.BlockSpec((Bn, U), lambda i: (i, 0)),
          pl.BlockSpec((Bn, U), lambda i: (i, 0)),
          pl.BlockSpec((2, Bn, U), lambda i: (0, i, 0)),
      ],
      out_shape=[
          jax.ShapeDtypeStruct((N, U), jnp.float32),
          jax.ShapeDtypeStruct((N, U), jnp.float32),
          jax.ShapeDtypeStruct((2, N, U), jnp.float32),
      ],
  )(nodes, sn, se, wft, b1f, w2f, b2f, wbt, b1b, w2b, b2b,
    wna, wnb, b1n, w2n, b2n, rs, fs, wpf, wpb)


def _node1_body(n_ref, r0_ref, sn_ref, se_ref,
                wft_ref, b1f_ref, w2f_ref, b2f_ref,
                wbt_ref, b1b_ref, w2b_ref, b2b_ref,
                wna_ref, wnb_ref, b1n_ref, w2n_ref, b2n_ref,
                rs_ref, wa_ref, ba_ref, wwt_ref, bw_ref, fs_ref,
                out_ref):
  x = n_ref[...]
  r0 = r0_ref[...]

  def msg(k, wt, b1, w2, b2):
    lin = jnp.dot(se_ref[k], wt[...],
                  preferred_element_type=jnp.float32) + sn_ref[k] + b1[...]
    h = jnp.maximum(lin, 0.0)
    return jnp.maximum(jnp.dot(h, w2[...],
                               preferred_element_type=jnp.float32) + b2[...], 0.0)

  comb = (msg(0, wft_ref, b1f_ref, w2f_ref, b2f_ref) +
          msg(1, wbt_ref, b1b_ref, w2b_ref, b2b_ref))
  h = jnp.maximum(
      jnp.dot(x, wna_ref[...], preferred_element_type=jnp.float32) +
      jnp.dot(comb, wnb_ref[...], preferred_element_type=jnp.float32) +
      b1n_ref[...], 0.0)
  nn = jnp.maximum(jnp.dot(h, w2n_ref[...],
                           preferred_element_type=jnp.float32) + b2n_ref[...], 0.0)
  r1 = _rms(nn, rs_ref[...])

  def att_w(r):
    a = jnp.tanh(jnp.dot(r, wa_ref[...],
                         preferred_element_type=jnp.float32) + ba_ref[...])
    return jnp.sum(a * wwt_ref[...], axis=-1, keepdims=True) + bw_ref[...]

  w0 = att_w(r0)
  w1 = att_w(r1)
  m = jnp.maximum(w0, w1)
  e0 = jnp.exp(w0 - m)
  e1 = jnp.exp(w1 - m)
  fused = (e0 * r0 + e1 * r1) / (e0 + e1)
  out_ref[...] = _rms(fused, fs_ref[...])


def _node_stage1(nodes1, repr0, sn, se, wft, b1f, w2f, b2f, wbt, b1b, w2b, b2b,
                 wna, wnb, b1n, w2n, b2n, rs, wa, ba, wwt, bw, fs):
  Bn = 1000
  grid = (N // Bn,)
  full = lambda i: (0, 0)
  U = UNITS
  return pl.pallas_call(
      _node1_body,
      grid=grid,
      in_specs=[
          pl.BlockSpec((Bn, U), lambda i: (i, 0)),
          pl.BlockSpec((Bn, U), lambda i: (i, 0)),
          pl.BlockSpec((2, Bn, U), lambda i: (0, i, 0)),
          pl.BlockSpec((2, Bn, U), lambda i: (0, i, 0)),
          pl.BlockSpec((U, U), full),
          pl.BlockSpec((1, U), full),
          pl.BlockSpec((U, U), full),
          pl.BlockSpec((1, U), full),
          pl.BlockSpec((U, U), full),
          pl.BlockSpec((1, U), full),
          pl.BlockSpec((U, U), full),
          pl.BlockSpec((1, U), full),
          pl.BlockSpec((U, U), full),
          pl.BlockSpec((U, U), full),
          pl.BlockSpec((1, U), full),
          pl.BlockSpec((U, U), full),
          pl.BlockSpec((1, U), full),
          pl.BlockSpec((1, U), full),
          pl.BlockSpec((U, U), full),
          pl.BlockSpec((1, U), full),
          pl.BlockSpec((1, U), full),
          pl.BlockSpec((1, 1), full),
          pl.BlockSpec((1, U), full),
      ],
      out_specs=pl.BlockSpec((Bn, U), lambda i: (i, 0)),
      out_shape=jax.ShapeDtypeStruct((N, U), jnp.float32),
  )(nodes1, repr0, sn, se, wft, b1f, w2f, b2f, wbt, b1b, w2b, b2b,
    wna, wnb, b1n, w2n, b2n, rs, wa, ba, wwt, bw, fs)


def _row(v):
  return v.reshape(1, -1)


def kernel(node_features, edge_features, edge_indices, edge_indices_reverse,
           params):
  lp0, lp1 = params["layers"]
  fs = _row(params["final_rms_scale"])

  dst_f = edge_indices[:, 0]
  src_f = edge_indices[:, 1]
  dst_b = edge_indices_reverse[:, 0]
  src_b = edge_indices_reverse[:, 1]
  src_off = jnp.concatenate([src_f, src_b + N])
  dst = jnp.concatenate([dst_f, dst_b])

  ef0, ef1 = _edge_prep(
      edge_features,
      lp0["edge"]["l1"]["W"], _row(lp0["edge"]["l1"]["b"]),
      lp0["edge"]["l2"]["W"], _row(lp0["edge"]["l2"]["b"]),
      _row(lp0["rms_scale"]))

  w1f0 = lp0["fwd"]["l1"]["W"]
  w1b0 = lp0["bwd"]["l1"]["W"]
  p0 = _proj(node_features, w1f0[D_EDGE:], w1b0[D_EDGE:]).reshape(2 * N, UNITS)

  sn0, se0 = _sc_segsum(D_EDGE, p0, ef0, src_off, dst)

  w1n0 = lp0["node"]["l1"]["W"]
  w1f1 = lp1["fwd"]["l1"]["W"]
  w1b1 = lp1["bwd"]["l1"]["W"]
  repr0, nodes1, p1 = _node_stage0(
      node_features,
      sn0, se0,
      w1f0[:D_EDGE], _row(lp0["fwd"]["l1"]["b"]),
      lp0["fwd"]["l2"]["W"], _row(lp0["fwd"]["l2"]["b"]),
      w1b0[:D_EDGE], _row(lp0["bwd"]["l1"]["b"]),
      lp0["bwd"]["l2"]["W"], _row(lp0["bwd"]["l2"]["b"]),
      w1n0[:D_NODE], w1n0[D_NODE:], _row(lp0["node"]["l1"]["b"]),
      lp0["node"]["l2"]["W"], _row(lp0["node"]["l2"]["b"]),
      _row(lp0["rms_scale"]), fs,
      w1f1[UNITS:], w1b1[UNITS:])

  sn1, se1 = _sc_segsum(UNITS, p1.reshape(2 * N, UNITS), ef1, src_off, dst)

  w1n1 = lp1["node"]["l1"]["W"]
  out = _node_stage1(
      nodes1, repr0,
      sn1, se1,
      w1f1[:UNITS], _row(lp1["fwd"]["l1"]["b"]),
      lp1["fwd"]["l2"]["W"], _row(lp1["fwd"]["l2"]["b"]),
      w1b1[:UNITS], _row(lp1["bwd"]["l1"]["b"]),
      lp1["bwd"]["l2"]["W"], _row(lp1["bwd"]["l2"]["b"]),
      w1n1[:UNITS], w1n1[UNITS:], _row(lp1["node"]["l1"]["b"]),
      lp1["node"]["l2"]["W"], _row(lp1["node"]["l2"]["b"]),
      _row(lp1["rms_scale"]),
      params["attn"]["W"], _row(params["attn"]["b"]),
      params["weights"]["W"].reshape(1, UNITS),
      params["weights"]["b"].reshape(1, 1),
      fs)
  return out


# speedup vs baseline: 7.5850x; 2.1527x over previous
import functools

import jax
import jax.numpy as jnp
from jax import lax
from jax.experimental import pallas as pl
from jax.experimental.pallas import tpu as pltpu
from jax.experimental.pallas import tpu_sc as plsc

N = 10000
E = 320000
D_NODE = 128
K = 2
D_EDGE = 16
ED_RAW = K * D_EDGE
UNITS = 64
EPS = 1e-6

CH = 128
NUM_CHUNKS = E // CH
NUM_TILES = 16
N_PAD = 10240
ROWS_PER_TILE = N_PAD // NUM_TILES
NB = 4
SLOTS_PER_TILE = 160
STAGE_SLOTS = 40
SLOTS = SLOTS_PER_TILE * NUM_TILES
E_PAD = SLOTS * CH
GROUPS = SLOTS_PER_TILE // NB


def _rms(x, scale):
  ms = jnp.mean(jnp.square(x), axis=-1, keepdims=True)
  return x * lax.rsqrt(ms + EPS) * scale


def _edge_mean_body(e_ref, out_ref):
  x = e_ref[...]
  out_ref[...] = 0.5 * (x[:, :D_EDGE] + x[:, D_EDGE:])


def _edge_mean(ef_raw):
  Br = 8000
  grid = (E // Br,)
  return pl.pallas_call(
      _edge_mean_body,
      grid=grid,
      in_specs=[pl.BlockSpec((Br, ED_RAW), lambda i: (i, 0))],
      out_specs=pl.BlockSpec((Br, D_EDGE), lambda i: (i, 0)),
      out_shape=jax.ShapeDtypeStruct((E, D_EDGE), jnp.float32),
      name="edge_mean",
  )(ef_raw)


def _edge_mlp_body(e_ref, w1_ref, b1_ref, w2_ref, b2_ref, rs_ref, out_ref):
  x = e_ref[...]
  acc = None
  for g in range(K):
    xg = x[:, D_EDGE * g:D_EDGE * (g + 1)]
    h = jnp.maximum(jnp.dot(xg, w1_ref[...],
                            preferred_element_type=jnp.float32) + b1_ref[...],
                    0.0)
    h = jnp.maximum(jnp.dot(h, w2_ref[...],
                            preferred_element_type=jnp.float32) + b2_ref[...],
                    0.0)
    r = _rms(h, rs_ref[...])
    acc = r if acc is None else acc + r
  out_ref[...] = 0.5 * acc


def _edge_mlp(ef_raw, w1, b1, w2, b2, rs):
  Br = 8000
  grid = (E // Br,)
  full = lambda i: (0, 0)
  return pl.pallas_call(
      _edge_mlp_body,
      grid=grid,
      in_specs=[
          pl.BlockSpec((Br, ED_RAW), lambda i: (i, 0)),
          pl.BlockSpec((D_EDGE, UNITS), full),
          pl.BlockSpec((1, UNITS), full),
          pl.BlockSpec((UNITS, UNITS), full),
          pl.BlockSpec((1, UNITS), full),
          pl.BlockSpec((1, UNITS), full),
      ],
      out_specs=pl.BlockSpec((Br, UNITS), lambda i: (i, 0)),
      out_shape=jax.ShapeDtypeStruct((E, UNITS), jnp.float32),
      name="edge_mlp",
  )(ef_raw, w1, b1, w2, b2, rs)


def _proj_body(n_ref, wf_ref, wb_ref, out_ref):
  x = n_ref[...]
  out_ref[0] = jnp.dot(x, wf_ref[...], preferred_element_type=jnp.float32)
  out_ref[1] = jnp.dot(x, wb_ref[...], preferred_element_type=jnp.float32)


def _proj(nodes, wf, wb):
  Bn = 1000
  grid = (N // Bn,)
  full = lambda i: (0, 0)
  return pl.pallas_call(
      _proj_body,
      grid=grid,
      in_specs=[
          pl.BlockSpec((Bn, D_NODE), lambda i: (i, 0)),
          pl.BlockSpec((D_NODE, UNITS), full),
          pl.BlockSpec((D_NODE, UNITS), full),
      ],
      out_specs=pl.BlockSpec((2, Bn, UNITS), lambda i: (0, i, 0)),
      out_shape=jax.ShapeDtypeStruct((2, N, UNITS), jnp.float32),
  )(nodes, wf, wb)


def _sc_body(ed, nb, p_hbm, ef_hbm, src_hbm, dst_hbm, zn_hbm, ze_hbm,
             out_n, out_e, acc_n, acc_e, srcbuf, dstbuf, *bufs):
  c = lax.axis_index("c")
  s = lax.axis_index("s")
  rows = list(bufs[0:nb])
  ebufs = list(bufs[nb:2 * nb])
  gsems = list(bufs[2 * nb:3 * nb])
  esems = list(bufs[3 * nb:4 * nb])

  r0 = s * ROWS_PER_TILE
  pltpu.sync_copy(zn_hbm.at[pl.ds(r0, ROWS_PER_TILE)],
                  acc_n.at[pl.ds(r0, ROWS_PER_TILE)])
  pltpu.sync_copy(ze_hbm.at[pl.ds(r0, ROWS_PER_TILE)],
                  acc_e.at[pl.ds(r0, ROWS_PER_TILE)])
  start = s * SLOTS_PER_TILE
  plsc.subcore_barrier()

  def run_stage(base):
    pltpu.sync_copy(src_hbm.at[c, pl.ds(start + base, STAGE_SLOTS)], srcbuf)
    pltpu.sync_copy(dst_hbm.at[c, pl.ds(start + base, STAGE_SLOTS)], dstbuf)

    def issue(i, b):
      j = start + base + i

      @pl.when(j < NUM_CHUNKS)
      def _():
        pltpu.async_copy(p_hbm.at[srcbuf.at[i]], rows[b], gsems[b])
        pltpu.async_copy(ef_hbm.at[pl.ds(j * CH, CH)], ebufs[b], esems[b])

    def drain(i, b):
      j = start + base + i

      @pl.when(j < NUM_CHUNKS)
      def _():
        pltpu.make_async_copy(p_hbm.at[srcbuf.at[i]], rows[b],
                              gsems[b]).wait()
        pltpu.sync_copy(rows[b], acc_n.at[dstbuf.at[i]], add=True)
        pltpu.make_async_copy(ef_hbm.at[pl.ds(j * CH, CH)], ebufs[b],
                              esems[b]).wait()
        pltpu.sync_copy(ebufs[b], acc_e.at[dstbuf.at[i]], add=True)

    for b in range(nb):
      issue(b, b)

    def group(g, _):
      for b in range(nb):
        i = g * nb + b
        drain(i, b)
        inx = i + nb

        @pl.when(inx < STAGE_SLOTS)
        def _():
          issue(inx, b)

      return ()

    lax.fori_loop(0, STAGE_SLOTS // nb, group, ())

  for base in range(0, SLOTS_PER_TILE, STAGE_SLOTS):
    run_stage(base)
  plsc.subcore_barrier()

  pltpu.sync_copy(acc_n.at[pl.ds(r0, ROWS_PER_TILE)],
                  out_n.at[pl.ds(c * N_PAD + r0, ROWS_PER_TILE)])
  pltpu.sync_copy(acc_e.at[pl.ds(r0, ROWS_PER_TILE)],
                  out_e.at[pl.ds(c * N_PAD + r0, ROWS_PER_TILE)])


def _sc_segsum(ed, nb, p_cat, ef, src_all, dst_all):
  mesh = plsc.VectorSubcoreMesh(core_axis_name="c", subcore_axis_name="s")
  zn = jnp.zeros((N_PAD, UNITS), jnp.float32)
  ze = jnp.zeros((N_PAD, ed), jnp.float32)
  run = pl.kernel(
      functools.partial(_sc_body, ed, nb),
      out_type=[
          jax.ShapeDtypeStruct((2 * N_PAD, UNITS), jnp.float32),
          jax.ShapeDtypeStruct((2 * N_PAD, ed), jnp.float32),
      ],
      mesh=mesh,
      scratch_types=[
          pltpu.VMEM_SHARED((N_PAD, UNITS), jnp.float32),
          pltpu.VMEM_SHARED((N_PAD, ed), jnp.float32),
          pltpu.VMEM((STAGE_SLOTS, CH), jnp.int32),
          pltpu.VMEM((STAGE_SLOTS, CH), jnp.int32),
      ] + [pltpu.VMEM((CH, UNITS), jnp.float32) for _ in range(nb)]
        + [pltpu.VMEM((CH, ed), jnp.float32) for _ in range(nb)]
        + [pltpu.SemaphoreType.DMA for _ in range(2 * nb)],
      compiler_params=pltpu.CompilerParams(use_tc_tiling_on_sc=False),
  )
  sn, se = run(p_cat, ef, src_all, dst_all, zn, ze)
  sn = sn.reshape(2, N_PAD, UNITS)[:, :N, :]
  se = se.reshape(2, N_PAD, ed)[:, :N, :]
  return sn, se


def _node0_body(n_ref, sn_ref, se_ref,
                wft_ref, b1f_ref, w2f_ref, b2f_ref,
                wbt_ref, b1b_ref, w2b_ref, b2b_ref,
                wna_ref, wnb_ref, b1n_ref, w2n_ref, b2n_ref,
                rs_ref, fs_ref, wpf_ref, wpb_ref,
                repr0_ref, nodes1_ref, p1_ref):
  x = n_ref[...]

  def msg(k, wt, b1, w2, b2):
    lin = jnp.dot(se_ref[k], wt[...],
                  preferred_element_type=jnp.float32) + sn_ref[k] + b1[...]
    h = jnp.maximum(lin, 0.0)
    return jnp.maximum(jnp.dot(h, w2[...],
                               preferred_element_type=jnp.float32) + b2[...], 0.0)

  comb = (msg(0, wft_ref, b1f_ref, w2f_ref, b2f_ref) +
          msg(1, wbt_ref, b1b_ref, w2b_ref, b2b_ref))
  h = jnp.maximum(
      jnp.dot(x, wna_ref[...], preferred_element_type=jnp.float32) +
      jnp.dot(comb, wnb_ref[...], preferred_element_type=jnp.float32) +
      b1n_ref[...], 0.0)
  nn = jnp.maximum(jnp.dot(h, w2n_ref[...],
                           preferred_element_type=jnp.float32) + b2n_ref[...], 0.0)
  r0 = _rms(nn, rs_ref[...])
  n1 = _rms(r0, fs_ref[...])
  repr0_ref[...] = r0
  nodes1_ref[...] = n1
  p1_ref[0] = jnp.dot(n1, wpf_ref[...], preferred_element_type=jnp.float32)
  p1_ref[1] = jnp.dot(n1, wpb_ref[...], preferred_element_type=jnp.float32)


def _node_stage0(nodes, sn, se, wft, b1f, w2f, b2f, wbt, b1b, w2b, b2b,
                 wna, wnb, b1n, w2n, b2n, rs, fs, wpf, wpb):
  Bn = 1000
  grid = (N // Bn,)
  full = lambda i: (0, 0)
  U = UNITS
  return pl.pallas_call(
      _node0_body,
      grid=grid,
      in_specs=[
          pl.BlockSpec((Bn, D_NODE), lambda i: (i, 0)),
          pl.BlockSpec((2, Bn, U), lambda i: (0, i, 0)),
          pl.BlockSpec((2, Bn, D_EDGE), lambda i: (0, i, 0)),
          pl.BlockSpec((D_EDGE, U), full),
          pl.BlockSpec((1, U), full),
          pl.BlockSpec((U, U), full),
          pl.BlockSpec((1, U), full),
          pl.BlockSpec((D_EDGE, U), full),
          pl.BlockSpec((1, U), full),
          pl.BlockSpec((U, U), full),
          pl.BlockSpec((1, U), full),
          pl.BlockSpec((D_NODE, U), full),
          pl.BlockSpec((U, U), full),
          pl.BlockSpec((1, U), full),
          pl.BlockSpec((U, U), full),
          pl.BlockSpec((1, U), full),
          pl.BlockSpec((1, U), full),
          pl.BlockSpec((1, U), full),
          pl.BlockSpec((U, U), full),
          pl.BlockSpec((U, U), full),
      ],
      out_specs=[
          pl.BlockSpec((Bn, U), lambda i: (i, 0)),
          pl.BlockSpec((Bn, U), lambda i: (i, 0)),
          pl.BlockSpec((2, Bn, U), lambda i: (0, i, 0)),
      ],
      out_shape=[
          jax.ShapeDtypeStruct((N, U), jnp.float32),
          jax.ShapeDtypeStruct((N, U), jnp.float32),
          jax.ShapeDtypeStruct((2, N, U), jnp.float32),
      ],
  )(nodes, sn, se, wft, b1f, w2f, b2f, wbt, b1b, w2b, b2b,
    wna, wnb, b1n, w2n, b2n, rs, fs, wpf, wpb)


def _node1_body(n_ref, r0_ref, sn_ref, se_ref,
                wft_ref, b1f_ref, w2f_ref, b2f_ref,
                wbt_ref, b1b_ref, w2b_ref, b2b_ref,
                wna_ref, wnb_ref, b1n_ref, w2n_ref, b2n_ref,
                rs_ref, wa_ref, ba_ref, wwt_ref, bw_ref, fs_ref,
                out_ref):
  x = n_ref[...]
  r0 = r0_ref[...]

  def msg(k, wt, b1, w2, b2):
    lin = jnp.dot(se_ref[k], wt[...],
                  preferred_element_type=jnp.float32) + sn_ref[k] + b1[...]
    h = jnp.maximum(lin, 0.0)
    return jnp.maximum(jnp.dot(h, w2[...],
                               preferred_element_type=jnp.float32) + b2[...], 0.0)

  comb = (msg(0, wft_ref, b1f_ref, w2f_ref, b2f_ref) +
          msg(1, wbt_ref, b1b_ref, w2b_ref, b2b_ref))
  h = jnp.maximum(
      jnp.dot(x, wna_ref[...], preferred_element_type=jnp.float32) +
      jnp.dot(comb, wnb_ref[...], preferred_element_type=jnp.float32) +
      b1n_ref[...], 0.0)
  nn = jnp.maximum(jnp.dot(h, w2n_ref[...],
                           preferred_element_type=jnp.float32) + b2n_ref[...], 0.0)
  r1 = _rms(nn, rs_ref[...])

  def att_w(r):
    a = jnp.tanh(jnp.dot(r, wa_ref[...],
                         preferred_element_type=jnp.float32) + ba_ref[...])
    return jnp.sum(a * wwt_ref[...], axis=-1, keepdims=True) + bw_ref[...]

  w0 = att_w(r0)
  w1 = att_w(r1)
  m = jnp.maximum(w0, w1)
  e0 = jnp.exp(w0 - m)
  e1 = jnp.exp(w1 - m)
  fused = (e0 * r0 + e1 * r1) / (e0 + e1)
  out_ref[...] = _rms(fused, fs_ref[...])


def _node_stage1(nodes1, repr0, sn, se, wft, b1f, w2f, b2f, wbt, b1b, w2b, b2b,
                 wna, wnb, b1n, w2n, b2n, rs, wa, ba, wwt, bw, fs):
  Bn = 1000
  grid = (N // Bn,)
  full = lambda i: (0, 0)
  U = UNITS
  return pl.pallas_call(
      _node1_body,
      grid=grid,
      in_specs=[
          pl.BlockSpec((Bn, U), lambda i: (i, 0)),
          pl.BlockSpec((Bn, U), lambda i: (i, 0)),
          pl.BlockSpec((2, Bn, U), lambda i: (0, i, 0)),
          pl.BlockSpec((2, Bn, U), lambda i: (0, i, 0)),
          pl.BlockSpec((U, U), full),
          pl.BlockSpec((1, U), full),
          pl.BlockSpec((U, U), full),
          pl.BlockSpec((1, U), full),
          pl.BlockSpec((U, U), full),
          pl.BlockSpec((1, U), full),
          pl.BlockSpec((U, U), full),
          pl.BlockSpec((1, U), full),
          pl.BlockSpec((U, U), full),
          pl.BlockSpec((U, U), full),
          pl.BlockSpec((1, U), full),
          pl.BlockSpec((U, U), full),
          pl.BlockSpec((1, U), full),
          pl.BlockSpec((1, U), full),
          pl.BlockSpec((U, U), full),
          pl.BlockSpec((1, U), full),
          pl.BlockSpec((1, U), full),
          pl.BlockSpec((1, 1), full),
          pl.BlockSpec((1, U), full),
      ],
      out_specs=pl.BlockSpec((Bn, U), lambda i: (i, 0)),
      out_shape=jax.ShapeDtypeStruct((N, U), jnp.float32),
  )(nodes1, repr0, sn, se, wft, b1f, w2f, b2f, wbt, b1b, w2b, b2b,
    wna, wnb, b1n, w2n, b2n, rs, wa, ba, wwt, bw, fs)


def _row(v):
  return v.reshape(1, -1)


def kernel(node_features, edge_features, edge_indices, edge_indices_reverse,
           params):
  lp0, lp1 = params["layers"]
  fs = _row(params["final_rms_scale"])

  def pack(col):
    pad = jnp.zeros((E_PAD - E,), jnp.int32)
    return jnp.concatenate([col, pad]).reshape(SLOTS, CH)

  src_all = jnp.stack([pack(edge_indices[:, 1]),
                       pack(edge_indices_reverse[:, 1] + N)])
  dst_all = jnp.stack([pack(edge_indices[:, 0]),
                       pack(edge_indices_reverse[:, 0])])

  ef_raw = edge_features.reshape(E, ED_RAW)
  w1f0 = lp0["fwd"]["l1"]["W"]
  w1b0 = lp0["bwd"]["l1"]["W"]
  p0 = _proj(node_features, w1f0[D_EDGE:], w1b0[D_EDGE:]).reshape(2 * N, UNITS)
  ef0 = _edge_mean(ef_raw)

  sn0, se0 = _sc_segsum(D_EDGE, 4, p0, ef0, src_all, dst_all)

  ef1 = _edge_mlp(
      ef_raw,
      lp0["edge"]["l1"]["W"], _row(lp0["edge"]["l1"]["b"]),
      lp0["edge"]["l2"]["W"], _row(lp0["edge"]["l2"]["b"]),
      _row(lp0["rms_scale"]))

  w1n0 = lp0["node"]["l1"]["W"]
  w1f1 = lp1["fwd"]["l1"]["W"]
  w1b1 = lp1["bwd"]["l1"]["W"]
  repr0, nodes1, p1 = _node_stage0(
      node_features,
      sn0, se0,
      w1f0[:D_EDGE], _row(lp0["fwd"]["l1"]["b"]),
      lp0["fwd"]["l2"]["W"], _row(lp0["fwd"]["l2"]["b"]),
      w1b0[:D_EDGE], _row(lp0["bwd"]["l1"]["b"]),
      lp0["bwd"]["l2"]["W"], _row(lp0["bwd"]["l2"]["b"]),
      w1n0[:D_NODE], w1n0[D_NODE:], _row(lp0["node"]["l1"]["b"]),
      lp0["node"]["l2"]["W"], _row(lp0["node"]["l2"]["b"]),
      _row(lp0["rms_scale"]), fs,
      w1f1[UNITS:], w1b1[UNITS:])

  sn1, se1 = _sc_segsum(UNITS, 2, p1.reshape(2 * N, UNITS), ef1,
                        src_all, dst_all)

  w1n1 = lp1["node"]["l1"]["W"]
  out = _node_stage1(
      nodes1, repr0,
      sn1, se1,
      w1f1[:UNITS], _row(lp1["fwd"]["l1"]["b"]),
      lp1["fwd"]["l2"]["W"], _row(lp1["fwd"]["l2"]["b"]),
      w1b1[:UNITS], _row(lp1["bwd"]["l1"]["b"]),
      lp1["bwd"]["l2"]["W"], _row(lp1["bwd"]["l2"]["b"]),
      w1n1[:UNITS], w1n1[UNITS:], _row(lp1["node"]["l1"]["b"]),
      lp1["node"]["l2"]["W"], _row(lp1["node"]["l2"]["b"]),
      _row(lp1["rms_scale"]),
      params["attn"]["W"], _row(params["attn"]["b"]),
      params["weights"]["W"].reshape(1, UNITS),
      params["weights"]["b"].reshape(1, 1),
      fs)
  return out


# speedup vs baseline: 8.3721x; 1.1038x over previous
import functools

import jax
import jax.numpy as jnp
from jax import lax
from jax.experimental import pallas as pl
from jax.experimental.pallas import tpu as pltpu
from jax.experimental.pallas import tpu_sc as plsc

N = 10000
E = 320000
D_NODE = 128
K = 2
D_EDGE = 16
ED_RAW = K * D_EDGE
UNITS = 64
EPS = 1e-6

CH = 128
NUM_CHUNKS = E // CH
NUM_TILES = 16
N_PAD = 10240
ROWS_PER_TILE = N_PAD // NUM_TILES
NB = 4
SLOTS_PER_TILE = 160
STAGE_SLOTS = 40
SLOTS = SLOTS_PER_TILE * NUM_TILES
E_PAD = SLOTS * CH
GROUPS = SLOTS_PER_TILE // NB


def _rms(x, scale):
  ms = jnp.mean(jnp.square(x), axis=-1, keepdims=True)
  return x * lax.rsqrt(ms + EPS) * scale


def _edge_mlp_body(e_ref, w1_ref, b1_ref, w2_ref, b2_ref, rs_ref, out_ref):
  x = e_ref[...]
  acc = None
  for g in range(K):
    xg = x[:, D_EDGE * g:D_EDGE * (g + 1)]
    h = jnp.maximum(jnp.dot(xg, w1_ref[...],
                            preferred_element_type=jnp.float32) + b1_ref[...],
                    0.0)
    h = jnp.maximum(jnp.dot(h, w2_ref[...],
                            preferred_element_type=jnp.float32) + b2_ref[...],
                    0.0)
    r = _rms(h, rs_ref[...])
    acc = r if acc is None else acc + r
  out_ref[...] = 0.5 * acc


def _edge_mlp(ef_raw, w1, b1, w2, b2, rs):
  Br = 8000
  grid = (E // Br,)
  full = lambda i: (0, 0)
  return pl.pallas_call(
      _edge_mlp_body,
      grid=grid,
      in_specs=[
          pl.BlockSpec((Br, ED_RAW), lambda i: (i, 0)),
          pl.BlockSpec((D_EDGE, UNITS), full),
          pl.BlockSpec((1, UNITS), full),
          pl.BlockSpec((UNITS, UNITS), full),
          pl.BlockSpec((1, UNITS), full),
          pl.BlockSpec((1, UNITS), full),
      ],
      out_specs=pl.BlockSpec((Br, UNITS), lambda i: (i, 0)),
      out_shape=jax.ShapeDtypeStruct((E, UNITS), jnp.float32),
      name="edge_mlp",
  )(ef_raw, w1, b1, w2, b2, rs)


def _proj_body(n_ref, wf_ref, wb_ref, out_ref):
  x = n_ref[...]
  out_ref[0] = jnp.dot(x, wf_ref[...], preferred_element_type=jnp.float32)
  out_ref[1] = jnp.dot(x, wb_ref[...], preferred_element_type=jnp.float32)


def _proj(nodes, wf, wb):
  Bn = 1000
  grid = (N // Bn,)
  full = lambda i: (0, 0)
  return pl.pallas_call(
      _proj_body,
      grid=grid,
      in_specs=[
          pl.BlockSpec((Bn, D_NODE), lambda i: (i, 0)),
          pl.BlockSpec((D_NODE, UNITS), full),
          pl.BlockSpec((D_NODE, UNITS), full),
      ],
      out_specs=pl.BlockSpec((2, Bn, UNITS), lambda i: (0, i, 0)),
      out_shape=jax.ShapeDtypeStruct((2, N, UNITS), jnp.float32),
  )(nodes, wf, wb)


def _sc_body(ed, nb, p_hbm, ef_hbm, src_hbm, dst_hbm, zn_hbm, ze_hbm,
             out_n, out_e, acc_n, acc_e, srcbuf, dstbuf, *bufs):
  c = lax.axis_index("c")
  s = lax.axis_index("s")
  rows = list(bufs[0:nb])
  ebufs = list(bufs[nb:2 * nb])
  gsems = list(bufs[2 * nb:3 * nb])
  esems = list(bufs[3 * nb:4 * nb])

  r0 = s * ROWS_PER_TILE
  pltpu.sync_copy(zn_hbm.at[pl.ds(r0, ROWS_PER_TILE)],
                  acc_n.at[pl.ds(r0, ROWS_PER_TILE)])
  pltpu.sync_copy(ze_hbm.at[pl.ds(r0, ROWS_PER_TILE)],
                  acc_e.at[pl.ds(r0, ROWS_PER_TILE)])
  start = s * SLOTS_PER_TILE
  plsc.subcore_barrier()

  def run_stage(base):
    pltpu.sync_copy(src_hbm.at[c, pl.ds(start + base, STAGE_SLOTS)], srcbuf)
    pltpu.sync_copy(dst_hbm.at[c, pl.ds(start + base, STAGE_SLOTS)], dstbuf)

    def issue(i, b):
      j = start + base + i

      @pl.when(j < NUM_CHUNKS)
      def _():
        pltpu.async_copy(p_hbm.at[srcbuf.at[i]], rows[b], gsems[b])
        pltpu.async_copy(ef_hbm.at[pl.ds(j * CH, CH)], ebufs[b], esems[b])

    def drain(i, b):
      j = start + base + i

      @pl.when(j < NUM_CHUNKS)
      def _():
        pltpu.make_async_copy(p_hbm.at[srcbuf.at[i]], rows[b],
                              gsems[b]).wait()
        pltpu.sync_copy(rows[b], acc_n.at[dstbuf.at[i]], add=True)
        pltpu.make_async_copy(ef_hbm.at[pl.ds(j * CH, CH)], ebufs[b],
                              esems[b]).wait()
        pltpu.sync_copy(ebufs[b], acc_e.at[dstbuf.at[i]], add=True)

    for b in range(nb):
      issue(b, b)

    def group(g, _):
      for b in range(nb):
        i = g * nb + b
        drain(i, b)
        inx = i + nb

        @pl.when(inx < STAGE_SLOTS)
        def _():
          issue(inx, b)

      return ()

    lax.fori_loop(0, STAGE_SLOTS // nb, group, ())

  for base in range(0, SLOTS_PER_TILE, STAGE_SLOTS):
    run_stage(base)
  plsc.subcore_barrier()

  pltpu.sync_copy(acc_n.at[pl.ds(r0, ROWS_PER_TILE)],
                  out_n.at[pl.ds(c * N_PAD + r0, ROWS_PER_TILE)])
  pltpu.sync_copy(acc_e.at[pl.ds(r0, ROWS_PER_TILE)],
                  out_e.at[pl.ds(c * N_PAD + r0, ROWS_PER_TILE)])


def _sc_segsum(ed, nb, p_cat, ef, src_all, dst_all):
  mesh = plsc.VectorSubcoreMesh(core_axis_name="c", subcore_axis_name="s")
  zn = jnp.zeros((N_PAD, UNITS), jnp.float32)
  ze = jnp.zeros((N_PAD, ed), jnp.float32)
  run = pl.kernel(
      functools.partial(_sc_body, ed, nb),
      out_type=[
          jax.ShapeDtypeStruct((2 * N_PAD, UNITS), jnp.float32),
          jax.ShapeDtypeStruct((2 * N_PAD, ed), jnp.float32),
      ],
      mesh=mesh,
      scratch_types=[
          pltpu.VMEM_SHARED((N_PAD, UNITS), jnp.float32),
          pltpu.VMEM_SHARED((N_PAD, ed), jnp.float32),
          pltpu.VMEM((STAGE_SLOTS, CH), jnp.int32),
          pltpu.VMEM((STAGE_SLOTS, CH), jnp.int32),
      ] + [pltpu.VMEM((CH, UNITS), jnp.float32) for _ in range(nb)]
        + [pltpu.VMEM((CH, ed), jnp.float32) for _ in range(nb)]
        + [pltpu.SemaphoreType.DMA for _ in range(2 * nb)],
      compiler_params=pltpu.CompilerParams(use_tc_tiling_on_sc=False),
  )
  sn, se = run(p_cat, ef, src_all, dst_all, zn, ze)
  sn = sn.reshape(2, N_PAD, UNITS)[:, :N, :]
  se = se.reshape(2, N_PAD, ed)[:, :N, :]
  return sn, se


def _node0_body(n_ref, sn_ref, se_ref,
                wft_ref, b1f_ref, w2f_ref, b2f_ref,
                wbt_ref, b1b_ref, w2b_ref, b2b_ref,
                wna_ref, wnb_ref, b1n_ref, w2n_ref, b2n_ref,
                rs_ref, fs_ref, wpf_ref, wpb_ref,
                repr0_ref, nodes1_ref, p1_ref):
  x = n_ref[...]

  def msg(k, wt, b1, w2, b2):
    se = 0.5 * (se_ref[k][:, :D_EDGE] + se_ref[k][:, D_EDGE:])
    lin = jnp.dot(se, wt[...],
                  preferred_element_type=jnp.float32) + sn_ref[k] + b1[...]
    h = jnp.maximum(lin, 0.0)
    return jnp.maximum(jnp.dot(h, w2[...],
                               preferred_element_type=jnp.float32) + b2[...], 0.0)

  comb = (msg(0, wft_ref, b1f_ref, w2f_ref, b2f_ref) +
          msg(1, wbt_ref, b1b_ref, w2b_ref, b2b_ref))
  h = jnp.maximum(
      jnp.dot(x, wna_ref[...], preferred_element_type=jnp.float32) +
      jnp.dot(comb, wnb_ref[...], preferred_element_type=jnp.float32) +
      b1n_ref[...], 0.0)
  nn = jnp.maximum(jnp.dot(h, w2n_ref[...],
                           preferred_element_type=jnp.float32) + b2n_ref[...], 0.0)
  r0 = _rms(nn, rs_ref[...])
  n1 = _rms(r0, fs_ref[...])
  repr0_ref[...] = r0
  nodes1_ref[...] = n1
  p1_ref[0] = jnp.dot(n1, wpf_ref[...], preferred_element_type=jnp.float32)
  p1_ref[1] = jnp.dot(n1, wpb_ref[...], preferred_element_type=jnp.float32)


def _node_stage0(nodes, sn, se, wft, b1f, w2f, b2f, wbt, b1b, w2b, b2b,
                 wna, wnb, b1n, w2n, b2n, rs, fs, wpf, wpb):
  Bn = 1000
  grid = (N // Bn,)
  full = lambda i: (0, 0)
  U = UNITS
  return pl.pallas_call(
      _node0_body,
      grid=grid,
      in_specs=[
          pl.BlockSpec((Bn, D_NODE), lambda i: (i, 0)),
          pl.BlockSpec((2, Bn, U), lambda i: (0, i, 0)),
          pl.BlockSpec((2, Bn, ED_RAW), lambda i: (0, i, 0)),
          pl.BlockSpec((D_EDGE, U), full),
          pl.BlockSpec((1, U), full),
          pl.BlockSpec((U, U), full),
          pl.BlockSpec((1, U), full),
          pl.BlockSpec((D_EDGE, U), full),
          pl.BlockSpec((1, U), full),
          pl.BlockSpec((U, U), full),
          pl.BlockSpec((1, U), full),
          pl.BlockSpec((D_NODE, U), full),
          pl.BlockSpec((U, U), full),
          pl.BlockSpec((1, U), full),
          pl.BlockSpec((U, U), full),
          pl.BlockSpec((1, U), full),
          pl.BlockSpec((1, U), full),
          pl.BlockSpec((1, U), full),
          pl.BlockSpec((U, U), full),
          pl.BlockSpec((U, U), full),
      ],
      out_specs=[
          pl.BlockSpec((Bn, U), lambda i: (i, 0)),
          pl.BlockSpec((Bn, U), lambda i: (i, 0)),
          pl.BlockSpec((2, Bn, U), lambda i: (0, i, 0)),
      ],
      out_shape=[
          jax.ShapeDtypeStruct((N, U), jnp.float32),
          jax.ShapeDtypeStruct((N, U), jnp.float32),
          jax.ShapeDtypeStruct((2, N, U), jnp.float32),
      ],
  )(nodes, sn, se, wft, b1f, w2f, b2f, wbt, b1b, w2b, b2b,
    wna, wnb, b1n, w2n, b2n, rs, fs, wpf, wpb)


def _node1_body(n_ref, r0_ref, sn_ref, se_ref,
                wft_ref, b1f_ref, w2f_ref, b2f_ref,
                wbt_ref, b1b_ref, w2b_ref, b2b_ref,
                wna_ref, wnb_ref, b1n_ref, w2n_ref, b2n_ref,
                rs_ref, wa_ref, ba_ref, wwt_ref, bw_ref, fs_ref,
                out_ref):
  x = n_ref[...]
  r0 = r0_ref[...]

  def msg(k, wt, b1, w2, b2):
    lin = jnp.dot(se_ref[k], wt[...],
                  preferred_element_type=jnp.float32) + sn_ref[k] + b1[...]
    h = jnp.maximum(lin, 0.0)
    return jnp.maximum(jnp.dot(h, w2[...],
                               preferred_element_type=jnp.float32) + b2[...], 0.0)

  comb = (msg(0, wft_ref, b1f_ref, w2f_ref, b2f_ref) +
          msg(1, wbt_ref, b1b_ref, w2b_ref, b2b_ref))
  h = jnp.maximum(
      jnp.dot(x, wna_ref[...], preferred_element_type=jnp.float32) +
      jnp.dot(comb, wnb_ref[...], preferred_element_type=jnp.float32) +
      b1n_ref[...], 0.0)
  nn = jnp.maximum(jnp.dot(h, w2n_ref[...],
                           preferred_element_type=jnp.float32) + b2n_ref[...], 0.0)
  r1 = _rms(nn, rs_ref[...])

  def att_w(r):
    a = jnp.tanh(jnp.dot(r, wa_ref[...],
                         preferred_element_type=jnp.float32) + ba_ref[...])
    return jnp.sum(a * wwt_ref[...], axis=-1, keepdims=True) + bw_ref[...]

  w0 = att_w(r0)
  w1 = att_w(r1)
  m = jnp.maximum(w0, w1)
  e0 = jnp.exp(w0 - m)
  e1 = jnp.exp(w1 - m)
  fused = (e0 * r0 + e1 * r1) / (e0 + e1)
  out_ref[...] = _rms(fused, fs_ref[...])


def _node_stage1(nodes1, repr0, sn, se, wft, b1f, w2f, b2f, wbt, b1b, w2b, b2b,
                 wna, wnb, b1n, w2n, b2n, rs, wa, ba, wwt, bw, fs):
  Bn = 1000
  grid = (N // Bn,)
  full = lambda i: (0, 0)
  U = UNITS
  return pl.pallas_call(
      _node1_body,
      grid=grid,
      in_specs=[
          pl.BlockSpec((Bn, U), lambda i: (i, 0)),
          pl.BlockSpec((Bn, U), lambda i: (i, 0)),
          pl.BlockSpec((2, Bn, U), lambda i: (0, i, 0)),
          pl.BlockSpec((2, Bn, U), lambda i: (0, i, 0)),
          pl.BlockSpec((U, U), full),
          pl.BlockSpec((1, U), full),
          pl.BlockSpec((U, U), full),
          pl.BlockSpec((1, U), full),
          pl.BlockSpec((U, U), full),
          pl.BlockSpec((1, U), full),
          pl.BlockSpec((U, U), full),
          pl.BlockSpec((1, U), full),
          pl.BlockSpec((U, U), full),
          pl.BlockSpec((U, U), full),
          pl.BlockSpec((1, U), full),
          pl.BlockSpec((U, U), full),
          pl.BlockSpec((1, U), full),
          pl.BlockSpec((1, U), full),
          pl.BlockSpec((U, U), full),
          pl.BlockSpec((1, U), full),
          pl.BlockSpec((1, U), full),
          pl.BlockSpec((1, 1), full),
          pl.BlockSpec((1, U), full),
      ],
      out_specs=pl.BlockSpec((Bn, U), lambda i: (i, 0)),
      out_shape=jax.ShapeDtypeStruct((N, U), jnp.float32),
  )(nodes1, repr0, sn, se, wft, b1f, w2f, b2f, wbt, b1b, w2b, b2b,
    wna, wnb, b1n, w2n, b2n, rs, wa, ba, wwt, bw, fs)


def _row(v):
  return v.reshape(1, -1)


def kernel(node_features, edge_features, edge_indices, edge_indices_reverse,
           params):
  lp0, lp1 = params["layers"]
  fs = _row(params["final_rms_scale"])

  def pack(col):
    pad = jnp.zeros((E_PAD - E,), jnp.int32)
    return jnp.concatenate([col, pad]).reshape(SLOTS, CH)

  src_all = jnp.stack([pack(edge_indices[:, 1]),
                       pack(edge_indices_reverse[:, 1] + N)])
  dst_all = jnp.stack([pack(edge_indices[:, 0]),
                       pack(edge_indices_reverse[:, 0])])

  ef_raw = edge_features.reshape(E, ED_RAW)
  w1f0 = lp0["fwd"]["l1"]["W"]
  w1b0 = lp0["bwd"]["l1"]["W"]
  p0 = _proj(node_features, w1f0[D_EDGE:], w1b0[D_EDGE:]).reshape(2 * N, UNITS)

  sn0, se0 = _sc_segsum(ED_RAW, 4, p0, ef_raw, src_all, dst_all)

  ef1 = _edge_mlp(
      ef_raw,
      lp0["edge"]["l1"]["W"], _row(lp0["edge"]["l1"]["b"]),
      lp0["edge"]["l2"]["W"], _row(lp0["edge"]["l2"]["b"]),
      _row(lp0["rms_scale"]))

  w1n0 = lp0["node"]["l1"]["W"]
  w1f1 = lp1["fwd"]["l1"]["W"]
  w1b1 = lp1["bwd"]["l1"]["W"]
  repr0, nodes1, p1 = _node_stage0(
      node_features,
      sn0, se0,
      w1f0[:D_EDGE], _row(lp0["fwd"]["l1"]["b"]),
      lp0["fwd"]["l2"]["W"], _row(lp0["fwd"]["l2"]["b"]),
      w1b0[:D_EDGE], _row(lp0["bwd"]["l1"]["b"]),
      lp0["bwd"]["l2"]["W"], _row(lp0["bwd"]["l2"]["b"]),
      w1n0[:D_NODE], w1n0[D_NODE:], _row(lp0["node"]["l1"]["b"]),
      lp0["node"]["l2"]["W"], _row(lp0["node"]["l2"]["b"]),
      _row(lp0["rms_scale"]), fs,
      w1f1[UNITS:], w1b1[UNITS:])

  sn1, se1 = _sc_segsum(UNITS, 2, p1.reshape(2 * N, UNITS), ef1,
                        src_all, dst_all)

  w1n1 = lp1["node"]["l1"]["W"]
  out = _node_stage1(
      nodes1, repr0,
      sn1, se1,
      w1f1[:UNITS], _row(lp1["fwd"]["l1"]["b"]),
      lp1["fwd"]["l2"]["W"], _row(lp1["fwd"]["l2"]["b"]),
      w1b1[:UNITS], _row(lp1["bwd"]["l1"]["b"]),
      lp1["bwd"]["l2"]["W"], _row(lp1["bwd"]["l2"]["b"]),
      w1n1[:UNITS], w1n1[UNITS:], _row(lp1["node"]["l1"]["b"]),
      lp1["node"]["l2"]["W"], _row(lp1["node"]["l2"]["b"]),
      _row(lp1["rms_scale"]),
      params["attn"]["W"], _row(params["attn"]["b"]),
      params["weights"]["W"].reshape(1, UNITS),
      params["weights"]["b"].reshape(1, 1),
      fs)
  return out
